# Initial kernel scaffold; baseline (speedup 1.0000x reference)
import jax
import jax.numpy as jnp
from jax.experimental import pallas as pl


def kernel(x, adj, W1, W2):
    raise NotImplementedError("write your pallas kernel here")


# speedup vs baseline: 6.2077x; 6.2077x over previous
import functools

import jax
import jax.numpy as jnp
from jax import lax
from jax.experimental import pallas as pl
from jax.experimental.pallas import tpu as pltpu
from jax.experimental.pallas import tpu_sc as plsc

N = 10000
E = 160000
D = 256
H = 256
C = 64
NP = 10240

NC = 2
NS = 16
CE = 128
RPT = NP // NS
NCH = 80
EPT_P = NCH * CE
NCH_D = 40
EPW_P = NCH_D * CE
DEG_W = 128

@functools.cache
def _get_deg_sc():
    mesh = plsc.VectorSubcoreMesh(core_axis_name="c", subcore_axis_name="s")

    @functools.partial(
        pl.kernel,
        out_type=jax.ShapeDtypeStruct((NC, NP, DEG_W), jnp.float32),
        mesh=mesh,
        scratch_types=[
            pltpu.VMEM((NCH_D, CE), jnp.int32),
            pltpu.VMEM((CE, DEG_W), jnp.float32),
            pltpu.VMEM_SHARED((NP, DEG_W), jnp.float32),
        ],
    )
    def deg_sc(dst_hbm, ones_hbm, zeros_hbm, out_hbm, dst_v, ones_v, acc):
        c = lax.axis_index("c")
        s = lax.axis_index("s")
        wid = c * NS + s
        pltpu.sync_copy(dst_hbm.at[wid], dst_v)
        pltpu.sync_copy(ones_hbm, ones_v)
        r0 = s * RPT
        pltpu.sync_copy(zeros_hbm.at[pl.ds(r0, RPT)], acc.at[pl.ds(r0, RPT)])
        plsc.subcore_barrier()

        def body(j, carry):
            pltpu.sync_copy(ones_v, acc.at[dst_v.at[j]], add=True)
            return carry

        lax.fori_loop(0, NCH_D, body, 0)
        plsc.subcore_barrier()
        pltpu.sync_copy(acc.at[pl.ds(r0, RPT)], out_hbm.at[c, pl.ds(r0, RPT)])

    return deg_sc


@functools.cache
def _make_prop(W):
    mesh = plsc.VectorSubcoreMesh(core_axis_name="c", subcore_axis_name="s")
    @functools.partial(
        pl.kernel,
        out_type=(
            jax.ShapeDtypeStruct((NP, W), jnp.float32),
            jax.ShapeDtypeStruct((NP, W), jnp.float32),
        ),
        mesh=mesh,
        scratch_types=[
            pltpu.VMEM((EPT_P,), jnp.int32),
            pltpu.VMEM((NCH, CE), jnp.int32),
            pltpu.VMEM((CE, W), jnp.float32),
            pltpu.VMEM_SHARED((NP, W), jnp.float32),
            pltpu.SemaphoreType.DMA,
        ],
    )
    def prop(z0, z1, src_hbm, dst_hbm, zeros_hbm, out0, out1,
             src_v, dst_v, buf, acc, sem):
        c = lax.axis_index("c")
        s = lax.axis_index("s")
        pltpu.sync_copy(src_hbm.at[s], src_v)
        pltpu.sync_copy(dst_hbm.at[s], dst_v)
        r0 = s * RPT
        pltpu.sync_copy(zeros_hbm.at[pl.ds(r0, RPT)], acc.at[pl.ds(r0, RPT)])
        plsc.subcore_barrier()

        def run(z_hbm, out_hbm):
            def body(j, carry):
                pltpu.async_copy(
                    z_hbm.at[src_v.at[pl.ds(j * CE, CE)]], buf, sem).wait()
                pltpu.sync_copy(buf, acc.at[dst_v.at[j]], add=True)
                return carry

            lax.fori_loop(0, NCH, body, 0)
            plsc.subcore_barrier()
            pltpu.sync_copy(acc.at[pl.ds(r0, RPT)],
                            out_hbm.at[pl.ds(r0, RPT)])

        @pl.when(c == 0)
        def _():
            run(z0, out0)

        @pl.when(c == 1)
        def _():
            run(z1, out1)

    return prop


@functools.cache
def _get_prop_es():
    mesh = plsc.VectorSubcoreMesh(core_axis_name="c", subcore_axis_name="s")

    @functools.partial(
        pl.kernel,
        out_type=jax.ShapeDtypeStruct((NC, NP, 128), jnp.float32),
        mesh=mesh,
        scratch_types=[
            pltpu.VMEM((EPW_P,), jnp.int32),
            pltpu.VMEM((NCH_D, CE), jnp.int32),
            pltpu.VMEM((CE, 128), jnp.float32),
            pltpu.VMEM_SHARED((NP, 128), jnp.float32),
            pltpu.SemaphoreType.DMA,
        ],
    )
    def prop_es(y_hbm, src_hbm, dst_hbm, zeros_hbm, out_hbm,
                src_v, dst_v, buf, acc, sem):
        c = lax.axis_index("c")
        s = lax.axis_index("s")
        wid = c * NS + s
        pltpu.sync_copy(src_hbm.at[wid], src_v)
        pltpu.sync_copy(dst_hbm.at[wid], dst_v)
        r0 = s * RPT
        pltpu.sync_copy(zeros_hbm.at[pl.ds(r0, RPT)], acc.at[pl.ds(r0, RPT)])
        plsc.subcore_barrier()

        def body(j, carry):
            pltpu.async_copy(
                y_hbm.at[src_v.at[pl.ds(j * CE, CE)]], buf, sem).wait()
            pltpu.sync_copy(buf, acc.at[dst_v.at[j]], add=True)
            return carry

        lax.fori_loop(0, NCH_D, body, 0)
        plsc.subcore_barrier()
        pltpu.sync_copy(acc.at[pl.ds(r0, RPT)], out_hbm.at[c, pl.ds(r0, RPT)])

    return prop_es


_RB = 1000


def _mm1_body(x_ref, w_ref, deg_ref, z0_ref, z1_ref, dinv_ref):
    deg = (deg_ref[0] + deg_ref[1])[:, :1]
    dinv = lax.rsqrt(jnp.maximum(deg, 1.0))
    z = jnp.dot(x_ref[...], w_ref[...], preferred_element_type=jnp.float32)
    z = z * dinv
    z0_ref[...] = z[:, : H // NC]
    z1_ref[...] = z[:, H // NC :]
    dinv_ref[...] = jnp.broadcast_to(dinv, (_RB, 128))


def _mm1(x, W1, deg_acc):
    return pl.pallas_call(
        _mm1_body,
        grid=(N // _RB,),
        in_specs=[
            pl.BlockSpec((_RB, D), lambda i: (i, 0)),
            pl.BlockSpec((D, H), lambda i: (0, 0)),
            pl.BlockSpec((NC, _RB, DEG_W), lambda i: (0, i, 0)),
        ],
        out_specs=[
            pl.BlockSpec((_RB, H // NC), lambda i: (i, 0)),
            pl.BlockSpec((_RB, H // NC), lambda i: (i, 0)),
            pl.BlockSpec((_RB, 128), lambda i: (i, 0)),
        ],
        out_shape=[
            jax.ShapeDtypeStruct((N, H // NC), jnp.float32),
            jax.ShapeDtypeStruct((N, H // NC), jnp.float32),
            jax.ShapeDtypeStruct((N, 128), jnp.float32),
        ],
    )(x, W1, deg_acc)


def _mm2_body(s10_ref, s11_ref, dinv_ref, w_ref, y_ref):
    dinv = dinv_ref[...]
    h0 = jnp.maximum(s10_ref[...] * dinv, 0.0)
    h1 = jnp.maximum(s11_ref[...] * dinv, 0.0)
    h = jnp.concatenate([h0, h1], axis=1)
    y = jnp.dot(h, w_ref[...], preferred_element_type=jnp.float32)
    y = y * dinv[:, :C]
    y_ref[...] = jnp.concatenate([y, jnp.zeros((_RB, 128 - C), y.dtype)],
                                 axis=1)


def _mm2(s10, s11, dinvb, W2):
    return pl.pallas_call(
        _mm2_body,
        grid=(N // _RB,),
        in_specs=[
            pl.BlockSpec((_RB, H // NC), lambda i: (i, 0)),
            pl.BlockSpec((_RB, H // NC), lambda i: (i, 0)),
            pl.BlockSpec((_RB, 128), lambda i: (i, 0)),
            pl.BlockSpec((H, C), lambda i: (0, 0)),
        ],
        out_specs=pl.BlockSpec((_RB, 128), lambda i: (i, 0)),
        out_shape=jax.ShapeDtypeStruct((N, 128), jnp.float32),
    )(s10, s11, dinvb, W2)


def _tail_body(p_ref, dinv_ref, out_ref):
    s2 = p_ref[0][:, :C] + p_ref[1][:, :C]
    out_ref[...] = s2 * dinv_ref[...][:, :C]


def _tail(parts, dinvb):
    return pl.pallas_call(
        _tail_body,
        grid=(N // _RB,),
        in_specs=[
            pl.BlockSpec((NC, _RB, 128), lambda i: (0, i, 0)),
            pl.BlockSpec((_RB, 128), lambda i: (i, 0)),
        ],
        out_specs=pl.BlockSpec((_RB, C), lambda i: (i, 0)),
        out_shape=jax.ShapeDtypeStruct((N, C), jnp.float32),
    )(parts, dinvb)


def kernel(x, adj, W1, W2):
    src = adj[0]
    dst = adj[1]
    ept = E // NS
    epw = E // (NC * NS)
    src_r = jnp.pad(src.reshape(NS, ept), ((0, 0), (0, EPT_P - ept)))
    dst_r = jnp.pad(dst.reshape(NS, ept), ((0, 0), (0, EPT_P - ept)),
                    constant_values=N).reshape(NS, NCH, CE)
    src_e = jnp.pad(src.reshape(NC * NS, epw), ((0, 0), (0, EPW_P - epw)))
    dst_e = jnp.pad(dst.reshape(NC * NS, epw), ((0, 0), (0, EPW_P - epw)),
                    constant_values=N).reshape(NC * NS, NCH_D, CE)
    ones_deg = jnp.ones((CE, DEG_W), jnp.float32)
    zeros_deg = jnp.zeros((NP, DEG_W), jnp.float32)
    zeros_h = jnp.zeros((NP, H // NC), jnp.float32)
    zeros_y = jnp.zeros((NP, 128), jnp.float32)

    deg_acc = _get_deg_sc()(dst_e, ones_deg, zeros_deg)
    z0, z1, dinvb = _mm1(x, W1, deg_acc)
    s10, s11 = _make_prop(128)(z0, z1, src_r, dst_r, zeros_h)
    y = _mm2(s10, s11, dinvb, W2)
    parts = _get_prop_es()(y, src_e, dst_e, zeros_y)
    return _tail(parts, dinvb)


# speedup vs baseline: 7.0060x; 1.1286x over previous
import functools

import jax
import jax.numpy as jnp
from jax import lax
from jax.experimental import pallas as pl
from jax.experimental.pallas import tpu as pltpu
from jax.experimental.pallas import tpu_sc as plsc

N = 10000
E = 160000
D = 256
H = 256
C = 64
NP = 10240

NC = 2
NS = 16
CE = 128
RPT = NP // NS
NCH = 80
EPT_P = NCH * CE
NCH_D = 40
EPW_P = NCH_D * CE
DEG_W = 128

@functools.cache
def _get_deg_sc():
    mesh = plsc.VectorSubcoreMesh(core_axis_name="c", subcore_axis_name="s")

    @functools.partial(
        pl.kernel,
        out_type=jax.ShapeDtypeStruct((NC, NP, DEG_W), jnp.float32),
        mesh=mesh,
        scratch_types=[
            pltpu.VMEM((NCH_D, CE), jnp.int32),
            pltpu.VMEM((CE, DEG_W), jnp.float32),
            pltpu.VMEM_SHARED((NP, DEG_W), jnp.float32),
        ],
    )
    def deg_sc(dst_hbm, ones_hbm, zeros_hbm, out_hbm, dst_v, ones_v, acc):
        c = lax.axis_index("c")
        s = lax.axis_index("s")
        wid = c * NS + s
        pltpu.sync_copy(dst_hbm.at[wid], dst_v)
        pltpu.sync_copy(ones_hbm, ones_v)
        r0 = s * RPT
        pltpu.sync_copy(zeros_hbm.at[pl.ds(r0, RPT)], acc.at[pl.ds(r0, RPT)])
        plsc.subcore_barrier()

        def body(j, carry):
            pltpu.sync_copy(ones_v, acc.at[dst_v.at[j]], add=True)
            return carry

        lax.fori_loop(0, NCH_D, body, 0)
        plsc.subcore_barrier()
        pltpu.sync_copy(acc.at[pl.ds(r0, RPT)], out_hbm.at[c, pl.ds(r0, RPT)])

    return deg_sc


@functools.cache
def _make_prop(W):
    mesh = plsc.VectorSubcoreMesh(core_axis_name="c", subcore_axis_name="s")
    @functools.partial(
        pl.kernel,
        out_type=(
            jax.ShapeDtypeStruct((NP, W), jnp.float32),
            jax.ShapeDtypeStruct((NP, W), jnp.float32),
        ),
        mesh=mesh,
        scratch_types=[
            pltpu.VMEM((NCH, CE), jnp.int32),
            pltpu.VMEM((8, CE), jnp.int32),
            pltpu.VMEM((8, CE), jnp.int32),
            pltpu.VMEM((CE, W), jnp.float32),
            pltpu.VMEM((CE, W), jnp.float32),
            pltpu.VMEM_SHARED((NP, W), jnp.float32),
            pltpu.SemaphoreType.DMA,
            pltpu.SemaphoreType.DMA,
            pltpu.SemaphoreType.DMA,
            pltpu.SemaphoreType.DMA,
        ],
    )
    def prop(z0, z1, src_hbm, dst_hbm, zeros_hbm, out0, out1,
             dst_v, sbuf_a, sbuf_b, buf_a, buf_b, acc, sia, sib, sa, sb):
        c = lax.axis_index("c")
        s = lax.axis_index("s")
        pltpu.sync_copy(dst_hbm.at[s], dst_v)
        r0 = s * RPT
        pltpu.sync_copy(zeros_hbm.at[pl.ds(r0, RPT)], acc.at[pl.ds(r0, RPT)])
        plsc.subcore_barrier()

        def run(z_hbm, out_hbm):
            pltpu.async_copy(src_hbm.at[s, pl.ds(0, CE)],
                             sbuf_a.at[0], sia).wait()
            pltpu.async_copy(z_hbm.at[sbuf_a.at[0]], buf_a, sa)
            pltpu.async_copy(src_hbm.at[s, pl.ds(CE, CE)], sbuf_b.at[0], sib)

            def idx_wait(sbuf, sem):
                pltpu.make_async_copy(src_hbm.at[s, pl.ds(0, CE)],
                                      sbuf.at[0], sem).wait()

            def gat_wait(buf, sem):
                pltpu.make_async_copy(z_hbm.at[sbuf_a.at[0]], buf, sem).wait()

            def body(i, carry):
                j = 2 * i
                idx_wait(sbuf_b, sib)
                gat_wait(buf_a, sa)
                pltpu.async_copy(z_hbm.at[sbuf_b.at[0]], buf_b, sb)
                pltpu.sync_copy(buf_a, acc.at[dst_v.at[j]], add=True)
                pltpu.async_copy(src_hbm.at[s, pl.ds((j + 2) * CE, CE)],
                                 sbuf_a.at[0], sia)
                gat_wait(buf_b, sb)
                idx_wait(sbuf_a, sia)
                pltpu.async_copy(z_hbm.at[sbuf_a.at[0]], buf_a, sa)
                pltpu.sync_copy(buf_b, acc.at[dst_v.at[j + 1]], add=True)
                pltpu.async_copy(src_hbm.at[s, pl.ds((j + 3) * CE, CE)],
                                 sbuf_b.at[0], sib)
                return carry

            lax.fori_loop(0, NCH // 2 - 1, body, 0)
            idx_wait(sbuf_b, sib)
            gat_wait(buf_a, sa)
            pltpu.async_copy(z_hbm.at[sbuf_b.at[0]], buf_b, sb)
            pltpu.sync_copy(buf_a, acc.at[dst_v.at[NCH - 2]], add=True)
            gat_wait(buf_b, sb)
            pltpu.sync_copy(buf_b, acc.at[dst_v.at[NCH - 1]], add=True)
            plsc.subcore_barrier()
            pltpu.sync_copy(acc.at[pl.ds(r0, RPT)],
                            out_hbm.at[pl.ds(r0, RPT)])

        @pl.when(c == 0)
        def _():
            run(z0, out0)

        @pl.when(c == 1)
        def _():
            run(z1, out1)

    return prop


@functools.cache
def _get_prop_es():
    mesh = plsc.VectorSubcoreMesh(core_axis_name="c", subcore_axis_name="s")

    @functools.partial(
        pl.kernel,
        out_type=jax.ShapeDtypeStruct((NC, NP, 128), jnp.float32),
        mesh=mesh,
        scratch_types=[
            pltpu.VMEM((EPW_P,), jnp.int32),
            pltpu.VMEM((NCH_D, CE), jnp.int32),
            pltpu.VMEM((CE, 128), jnp.float32),
            pltpu.VMEM((CE, 128), jnp.float32),
            pltpu.VMEM_SHARED((NP, 128), jnp.float32),
            pltpu.SemaphoreType.DMA,
            pltpu.SemaphoreType.DMA,
        ],
    )
    def prop_es(y_hbm, src_hbm, dst_hbm, zeros_hbm, out_hbm,
                src_v, dst_v, buf_a, buf_b, acc, sa, sb):
        c = lax.axis_index("c")
        s = lax.axis_index("s")
        wid = c * NS + s
        pltpu.sync_copy(src_hbm.at[wid], src_v)
        pltpu.sync_copy(dst_hbm.at[wid], dst_v)
        r0 = s * RPT
        pltpu.sync_copy(zeros_hbm.at[pl.ds(r0, RPT)], acc.at[pl.ds(r0, RPT)])
        plsc.subcore_barrier()

        def gather(j, buf, sem):
            return pltpu.async_copy(
                y_hbm.at[src_v.at[pl.ds(j * CE, CE)]], buf, sem)

        def gat_wait(buf, sem):
            pltpu.make_async_copy(y_hbm.at[src_v.at[pl.ds(0, CE)]],
                                  buf, sem).wait()

        gather(0, buf_a, sa)

        def body(i, carry):
            j = 2 * i
            gather(j + 1, buf_b, sb)
            gat_wait(buf_a, sa)
            pltpu.sync_copy(buf_a, acc.at[dst_v.at[j]], add=True)
            gather(j + 2, buf_a, sa)
            gat_wait(buf_b, sb)
            pltpu.sync_copy(buf_b, acc.at[dst_v.at[j + 1]], add=True)
            return carry

        lax.fori_loop(0, NCH_D // 2 - 1, body, 0)
        gather(NCH_D - 1, buf_b, sb)
        gat_wait(buf_a, sa)
        pltpu.sync_copy(buf_a, acc.at[dst_v.at[NCH_D - 2]], add=True)
        gat_wait(buf_b, sb)
        pltpu.sync_copy(buf_b, acc.at[dst_v.at[NCH_D - 1]], add=True)
        plsc.subcore_barrier()
        pltpu.sync_copy(acc.at[pl.ds(r0, RPT)], out_hbm.at[c, pl.ds(r0, RPT)])

    return prop_es


_RB = 1000


def _mm1_body(x_ref, w_ref, deg_ref, z0_ref, z1_ref, dinv_ref):
    deg = (deg_ref[0] + deg_ref[1])[:, :1]
    dinv = lax.rsqrt(jnp.maximum(deg, 1.0))
    z = jnp.dot(x_ref[...], w_ref[...], preferred_element_type=jnp.float32)
    z = z * dinv
    z0_ref[...] = z[:, : H // NC]
    z1_ref[...] = z[:, H // NC :]
    dinv_ref[...] = jnp.broadcast_to(dinv, (_RB, 128))


def _mm1(x, W1, deg_acc):
    return pl.pallas_call(
        _mm1_body,
        grid=(N // _RB,),
        in_specs=[
            pl.BlockSpec((_RB, D), lambda i: (i, 0)),
            pl.BlockSpec((D, H), lambda i: (0, 0)),
            pl.BlockSpec((NC, _RB, DEG_W), lambda i: (0, i, 0)),
        ],
        out_specs=[
            pl.BlockSpec((_RB, H // NC), lambda i: (i, 0)),
            pl.BlockSpec((_RB, H // NC), lambda i: (i, 0)),
            pl.BlockSpec((_RB, 128), lambda i: (i, 0)),
        ],
        out_shape=[
            jax.ShapeDtypeStruct((N, H // NC), jnp.float32),
            jax.ShapeDtypeStruct((N, H // NC), jnp.float32),
            jax.ShapeDtypeStruct((N, 128), jnp.float32),
        ],
    )(x, W1, deg_acc)


def _mm2_body(s10_ref, s11_ref, dinv_ref, w_ref, y_ref):
    dinv = dinv_ref[...]
    h0 = jnp.maximum(s10_ref[...] * dinv, 0.0)
    h1 = jnp.maximum(s11_ref[...] * dinv, 0.0)
    h = jnp.concatenate([h0, h1], axis=1)
    y = jnp.dot(h, w_ref[...], preferred_element_type=jnp.float32)
    y = y * dinv[:, :C]
    y_ref[...] = jnp.concatenate([y, jnp.zeros((_RB, 128 - C), y.dtype)],
                                 axis=1)


def _mm2(s10, s11, dinvb, W2):
    return pl.pallas_call(
        _mm2_body,
        grid=(N // _RB,),
        in_specs=[
            pl.BlockSpec((_RB, H // NC), lambda i: (i, 0)),
            pl.BlockSpec((_RB, H // NC), lambda i: (i, 0)),
            pl.BlockSpec((_RB, 128), lambda i: (i, 0)),
            pl.BlockSpec((H, C), lambda i: (0, 0)),
        ],
        out_specs=pl.BlockSpec((_RB, 128), lambda i: (i, 0)),
        out_shape=jax.ShapeDtypeStruct((N, 128), jnp.float32),
    )(s10, s11, dinvb, W2)


def _tail_body(p_ref, dinv_ref, out_ref):
    s2 = p_ref[0][:, :C] + p_ref[1][:, :C]
    out_ref[...] = s2 * dinv_ref[...][:, :C]


def _tail(parts, dinvb):
    return pl.pallas_call(
        _tail_body,
        grid=(N // _RB,),
        in_specs=[
            pl.BlockSpec((NC, _RB, 128), lambda i: (0, i, 0)),
            pl.BlockSpec((_RB, 128), lambda i: (i, 0)),
        ],
        out_specs=pl.BlockSpec((_RB, C), lambda i: (i, 0)),
        out_shape=jax.ShapeDtypeStruct((N, C), jnp.float32),
    )(parts, dinvb)


def kernel(x, adj, W1, W2):
    src = adj[0]
    dst = adj[1]
    ept = E // NS
    epw = E // (NC * NS)
    src_r = jnp.pad(src.reshape(NS, ept), ((0, 0), (0, EPT_P - ept)))
    dst_r = jnp.pad(dst.reshape(NS, ept), ((0, 0), (0, EPT_P - ept)),
                    constant_values=N).reshape(NS, NCH, CE)
    src_e = jnp.pad(src.reshape(NC * NS, epw), ((0, 0), (0, EPW_P - epw)))
    dst_e = jnp.pad(dst.reshape(NC * NS, epw), ((0, 0), (0, EPW_P - epw)),
                    constant_values=N).reshape(NC * NS, NCH_D, CE)
    ones_deg = jnp.ones((CE, DEG_W), jnp.float32)
    zeros_deg = jnp.zeros((NP, DEG_W), jnp.float32)
    zeros_h = jnp.zeros((NP, H // NC), jnp.float32)
    zeros_y = jnp.zeros((NP, 128), jnp.float32)

    deg_acc = _get_deg_sc()(dst_e, ones_deg, zeros_deg)
    z0, z1, dinvb = _mm1(x, W1, deg_acc)
    s10, s11 = _make_prop(128)(z0, z1, src_r, dst_r, zeros_h)
    y = _mm2(s10, s11, dinvb, W2)
    parts = _get_prop_es()(y, src_e, dst_e, zeros_y)
    return _tail(parts, dinvb)
